# Initial kernel scaffold; baseline (speedup 1.0000x reference)
import jax
import jax.numpy as jnp
from jax.experimental import pallas as pl


def kernel(x, edge_index, batch, params):
    raise NotImplementedError("write your pallas kernel here")


# speedup vs baseline: 4.1995x; 4.1995x over previous
import functools

import jax
import jax.numpy as jnp
from jax import lax
from jax.experimental import pallas as pl
from jax.experimental.pallas import tpu as pltpu
from jax.experimental.pallas import tpu_sc as plsc

N = 10000
E = 320000
D = 128
H = 128
OUT = 10
G = 64

NC = 2
NS = 16
NW = NC * NS

N_PAD = 10240
K = 79
E_PAD = NW * K * 128
RPS = N_PAD // NS

_PREC = lax.Precision.HIGHEST


def _agg_body(h_hbm, src_hbm, dst_hbm, zeros_hbm, out_hbm,
              src_v, dst_v, rows_v, acc_sh, sem):
    c = lax.axis_index("c")
    s = lax.axis_index("s")
    w = s * NC + c

    pltpu.sync_copy(zeros_hbm, acc_sh.at[pl.ds(s * RPS, RPS)])

    pltpu.sync_copy(src_hbm.at[w], src_v)
    pltpu.sync_copy(dst_hbm.at[w], dst_v)

    plsc.subcore_barrier()

    def body(j, carry):
        pltpu.async_copy(h_hbm.at[src_v.at[j]], rows_v, sem).wait()
        pltpu.sync_copy(rows_v, acc_sh.at[dst_v.at[j]], add=True)
        return carry

    lax.fori_loop(0, K, body, 0)

    plsc.subcore_barrier()

    pltpu.sync_copy(acc_sh.at[pl.ds(s * RPS, RPS)],
                    out_hbm.at[c, pl.ds(s * RPS, RPS)])


@functools.cache
def _make_agg():
    return pl.kernel(
        _agg_body,
        out_type=jax.ShapeDtypeStruct((NC, N_PAD, H), jnp.float32),
        mesh=plsc.VectorSubcoreMesh(core_axis_name="c", subcore_axis_name="s",
                                    num_cores=NC, num_subcores=NS),
        scratch_types=[
            pltpu.VMEM((K, 128), jnp.int32),
            pltpu.VMEM((K, 128), jnp.int32),
            pltpu.VMEM((128, H), jnp.float32),
            pltpu.VMEM_SHARED((N_PAD, H), jnp.float32),
            pltpu.SemaphoreType.DMA,
        ],
    )


def _agg(h, src_p, dst_p, zeros_blk):
    return _make_agg()(h, src_p, dst_p, zeros_blk)


def _layer_body(h_ref, agg_ref, W1_ref, b1_ref, g1_ref, be1_ref,
                W2_ref, b2_ref, g2_ref, be2_ref, o_ref):
    mf = (lax.broadcasted_iota(jnp.int32, (N_PAD, 1), 0) < N).astype(
        jnp.float32)
    inv_n = 1.0 / N

    h = h_ref[...] + agg_ref[0] + agg_ref[1]

    h = jnp.dot(h, W1_ref[...], precision=_PREC) + b1_ref[...]
    mu = jnp.sum(h * mf, axis=0, keepdims=True) * inv_n
    d = h - mu
    var = jnp.sum(d * d * mf, axis=0, keepdims=True) * inv_n
    h = g1_ref[...] * d * lax.rsqrt(var + 1e-5) + be1_ref[...]
    h = jnp.maximum(h, 0.0) * mf

    h = jnp.dot(h, W2_ref[...], precision=_PREC) + b2_ref[...]
    mu = jnp.sum(h * mf, axis=0, keepdims=True) * inv_n
    d = h - mu
    var = jnp.sum(d * d * mf, axis=0, keepdims=True) * inv_n
    h = g2_ref[...] * d * lax.rsqrt(var + 1e-5) + be2_ref[...]
    o_ref[...] = jnp.maximum(h, 0.0) * mf


_layer = pl.pallas_call(
    _layer_body,
    out_shape=jax.ShapeDtypeStruct((N_PAD, H), jnp.float32),
)


def _head_body(b_ref, h0_ref, h1_ref, h2_ref, h3_ref, h4_ref, h5_ref,
               W1_ref, b1_ref, W3_ref, b3_ref, o_ref):
    gid = lax.broadcasted_iota(jnp.int32, (1, G), 1)
    oh = (b_ref[...] == gid).astype(jnp.float32)
    dn = (((0,), (0,)), ((), ()))
    sums = [
        lax.dot_general(oh, r[...], dn, precision=_PREC)
        for r in (h0_ref, h1_ref, h2_ref, h3_ref, h4_ref, h5_ref)
    ]
    cnt = lax.dot_general(oh, jnp.ones((N_PAD, 1), jnp.float32), dn,
                          precision=_PREC)
    hg = jnp.concatenate(sums, axis=1) / jnp.maximum(cnt, 1.0)
    z = jnp.maximum(jnp.dot(hg, W1_ref[...], precision=_PREC) + b1_ref[...],
                    0.0)
    o_ref[...] = jnp.dot(z, W3_ref[...], precision=_PREC) + b3_ref[...]


_head = pl.pallas_call(
    _head_body,
    out_shape=jax.ShapeDtypeStruct((G, OUT), jnp.float32),
)


def kernel(x, edge_index, batch, params):
    src = edge_index[0].astype(jnp.int32)
    dst = edge_index[1].astype(jnp.int32)
    pad = jnp.full((E_PAD - E,), N, jnp.int32)
    src_p = jnp.concatenate([src, pad]).reshape(NW, K, 128)
    dst_p = jnp.concatenate([dst, pad]).reshape(NW, K, 128)
    zeros_blk = jnp.zeros((RPS, H), jnp.float32)
    batch_p = jnp.concatenate(
        [batch.astype(jnp.int32), jnp.full((N_PAD - N,), G, jnp.int32)]
    ).reshape(N_PAD, 1)

    h = jnp.zeros((N_PAD, D), x.dtype).at[:N].set(x)
    hs = [h]
    for i in range(1, 6):
        parts = _agg(h, src_p, dst_p, zeros_blk)
        h = _layer(
            h, parts,
            params["conv%d_W1" % i], params["conv%d_b1" % i].reshape(1, H),
            params["conv%d_bn_g" % i].reshape(1, H),
            params["conv%d_bn_b" % i].reshape(1, H),
            params["conv%d_W2" % i], params["conv%d_b2" % i].reshape(1, H),
            params["norm%d_g" % i].reshape(1, H),
            params["norm%d_b" % i].reshape(1, H),
        )
        hs.append(h)

    return _head(
        batch_p, *hs,
        params["fc1_W"], params["fc1_b"].reshape(1, -1),
        params["fc3_W"], params["fc3_b"].reshape(1, -1),
    )
